# Initial kernel scaffold; baseline (speedup 1.0000x reference)
import jax
import jax.numpy as jnp
from jax.experimental import pallas as pl


def kernel(position, table):
    raise NotImplementedError("write your pallas kernel here")


# speedup vs baseline: 1.4828x; 1.4828x over previous
import jax
import jax.numpy as jnp
from jax import lax
from jax.experimental import pallas as pl
from jax.experimental.pallas import tpu as pltpu
from jax.experimental.pallas import tpu_sc as plsc

BLOCK = 8192
EMBD = 1024
NC = 2
NS = 16
NW = NC * NS
BPW = BLOCK // NW
CHUNK = 32
NCHUNK = BPW // CHUNK


def _body(pos_hbm, table_hbm, out_hbm, idx_v, buf0, buf1, gsem0, gsem1,
          osem0, osem1):
    wid = lax.axis_index("s") * NC + lax.axis_index("c")
    base = wid * BPW
    pltpu.sync_copy(pos_hbm.at[pl.ds(base, BPW)], idx_v)

    bufs = (buf0, buf1)
    gsems = (gsem0, gsem1)
    osems = (osem0, osem1)

    def start_gather(c):
        return pltpu.async_copy(
            table_hbm.at[idx_v.at[pl.ds(c * CHUNK, CHUNK)]],
            bufs[c & 1], gsems[c & 1])

    out_copies = [None] * NCHUNK
    gathers = [None] * NCHUNK
    gathers[0] = start_gather(0)
    for c in range(NCHUNK):
        b = c & 1
        gathers[c].wait()
        if c >= 1:
            out_copies[c - 1].wait()
        if c + 1 < NCHUNK:
            gathers[c + 1] = start_gather(c + 1)
        out_copies[c] = pltpu.async_copy(
            bufs[b], out_hbm.at[pl.ds(base + c * CHUNK, CHUNK)], osems[b])
    out_copies[NCHUNK - 1].wait()


def kernel(position, table):
    run = pl.kernel(
        _body,
        out_type=jax.ShapeDtypeStruct((BLOCK, EMBD), jnp.float32),
        mesh=plsc.VectorSubcoreMesh(core_axis_name="c", subcore_axis_name="s"),
        scratch_types=[
            pltpu.VMEM((BPW,), jnp.int32),
            pltpu.VMEM((CHUNK, EMBD), jnp.float32),
            pltpu.VMEM((CHUNK, EMBD), jnp.float32),
            pltpu.SemaphoreType.DMA,
            pltpu.SemaphoreType.DMA,
            pltpu.SemaphoreType.DMA,
            pltpu.SemaphoreType.DMA,
        ],
    )
    return run(position.astype(jnp.int32), table)
